# Initial kernel scaffold; baseline (speedup 1.0000x reference)
import jax
import jax.numpy as jnp
from jax.experimental import pallas as pl


def kernel(h, edge_index, W1, b1, W2, b2):
    raise NotImplementedError("write your pallas kernel here")


# speedup vs baseline: 3.8966x; 3.8966x over previous
import functools

import jax
import jax.numpy as jnp
from jax import lax
from jax.experimental import pallas as pl
from jax.experimental.pallas import tpu as pltpu
from jax.experimental.pallas import tpu_sc as plsc

N_NODES = 10000
N_EDGES = 320000
D_IN = 128
D_HID = 128
N_CLS = 40
N_CLS_PAD = 48

NC = 2
NS = 16
NW = NC * NS
BLK = 128
NB = 80
E_PAD = NW * NB * BLK
N_BLOCKS_TOTAL = NW * NB
N_PAD = 10240
ROWS_PER_TILE = N_PAD // NS

_MESH = plsc.VectorSubcoreMesh(core_axis_name="c", subcore_axis_name="s")


@functools.partial(
    pl.kernel,
    out_type=(
        jax.ShapeDtypeStruct((NC, N_PAD), jnp.float32),
        jax.ShapeDtypeStruct((NC, N_PAD), jnp.float32),
    ),
    mesh=_MESH,
    scratch_types=[
        pltpu.VMEM((NB, BLK), jnp.int32),
        pltpu.VMEM((NB, BLK), jnp.int32),
        pltpu.VMEM((BLK,), jnp.float32),
        pltpu.VMEM_SHARED((N_PAD,), jnp.float32),
        pltpu.VMEM_SHARED((N_PAD,), jnp.float32),
    ],
)
def _sc_degrees(src_hbm, dst_hbm, zeros_hbm, dsrc_out, ddst_out,
                sidx, didx, ones, acc_s, acc_d):
    c = lax.axis_index("c")
    s = lax.axis_index("s")
    w = c * NS + s
    for k in range(BLK // 16):
        ones[pl.ds(k * 16, 16)] = jnp.ones((16,), jnp.float32)
    r0 = s * ROWS_PER_TILE
    pltpu.sync_copy(zeros_hbm.at[pl.ds(r0, ROWS_PER_TILE)],
                    acc_s.at[pl.ds(r0, ROWS_PER_TILE)])
    pltpu.sync_copy(zeros_hbm.at[pl.ds(r0, ROWS_PER_TILE)],
                    acc_d.at[pl.ds(r0, ROWS_PER_TILE)])
    pltpu.sync_copy(src_hbm.at[pl.ds(w * NB, NB)], sidx)
    pltpu.sync_copy(dst_hbm.at[pl.ds(w * NB, NB)], didx)
    plsc.subcore_barrier()

    def body(j, carry):
        pltpu.sync_copy(ones, acc_s.at[sidx.at[j]], add=True)
        pltpu.sync_copy(ones, acc_d.at[didx.at[j]], add=True)
        return carry

    lax.fori_loop(0, NB, body, 0)
    plsc.subcore_barrier()
    pltpu.sync_copy(acc_s.at[pl.ds(r0, ROWS_PER_TILE)],
                    dsrc_out.at[c, pl.ds(r0, ROWS_PER_TILE)])
    pltpu.sync_copy(acc_d.at[pl.ds(r0, ROWS_PER_TILE)],
                    ddst_out.at[c, pl.ds(r0, ROWS_PER_TILE)])


def _make_sc_aggregate(d_width):
    @functools.partial(
        pl.kernel,
        out_type=jax.ShapeDtypeStruct((NC, N_PAD, d_width), jnp.float32),
        mesh=_MESH,
        scratch_types=[
            pltpu.VMEM((NB, BLK), jnp.int32),
            pltpu.VMEM((NB, BLK), jnp.int32),
            pltpu.VMEM((BLK, d_width), jnp.float32),
            pltpu.VMEM_SHARED((N_PAD, d_width), jnp.float32),
            pltpu.SemaphoreType.DMA,
        ],
        compiler_params=pltpu.CompilerParams(
            use_tc_tiling_on_sc=(d_width % 128 == 0)),
    )
    def _sc_aggregate(vals_hbm, src_hbm, dst_hbm, zeros_hbm, out,
                      sidx, didx, buf, acc, sem):
        c = lax.axis_index("c")
        s = lax.axis_index("s")
        w = c * NS + s
        r0 = s * ROWS_PER_TILE
        pltpu.sync_copy(zeros_hbm.at[pl.ds(r0, ROWS_PER_TILE)],
                        acc.at[pl.ds(r0, ROWS_PER_TILE)])
        pltpu.sync_copy(src_hbm.at[pl.ds(w * NB, NB)], sidx)
        pltpu.sync_copy(dst_hbm.at[pl.ds(w * NB, NB)], didx)
        plsc.subcore_barrier()

        def body(j, carry):
            pltpu.async_copy(vals_hbm.at[sidx.at[j]], buf, sem).wait()
            pltpu.sync_copy(buf, acc.at[didx.at[j]], add=True)
            return carry

        lax.fori_loop(0, NB, body, 0)
        plsc.subcore_barrier()
        pltpu.sync_copy(acc.at[pl.ds(r0, ROWS_PER_TILE)],
                        out.at[c, pl.ds(r0, ROWS_PER_TILE)])

    return _sc_aggregate


_sc_aggregate_128 = _make_sc_aggregate(D_HID)
_sc_aggregate_48 = _make_sc_aggregate(N_CLS_PAD)


_BM = 1024
_GRID = N_PAD // _BM


def _norm_col(deg2):
    d = deg2[:, 0:1] + deg2[:, 1:2]
    return jnp.where(d > 0.0, lax.rsqrt(d), 0.0)


def _tc_pre_body(h_ref, dsrc_ref, w1_ref, o_ref):
    x = h_ref[...] * _norm_col(dsrc_ref[...])
    o_ref[...] = jnp.dot(x, w1_ref[...], preferred_element_type=jnp.float32)


def _tc_pre(h_p, dsrc_t, W1):
    return pl.pallas_call(
        _tc_pre_body,
        grid=(_GRID,),
        in_specs=[
            pl.BlockSpec((_BM, D_IN), lambda i: (i, 0)),
            pl.BlockSpec((_BM, NC), lambda i: (i, 0)),
            pl.BlockSpec((D_IN, D_HID), lambda i: (0, 0)),
        ],
        out_specs=pl.BlockSpec((_BM, D_HID), lambda i: (i, 0)),
        out_shape=jax.ShapeDtypeStruct((N_PAD, D_HID), jnp.float32),
    )(h_p, dsrc_t, W1)


def _tc_mid_body(a_ref, din_ref, dsrc_ref, b1_ref, w2_ref, o_ref):
    agg = a_ref[0] + a_ref[1]
    h1 = jnp.maximum(agg * _norm_col(din_ref[...]) + b1_ref[...], 0.0)
    x = h1 * _norm_col(dsrc_ref[...])
    o_ref[...] = jnp.dot(x, w2_ref[...], preferred_element_type=jnp.float32)


def _tc_mid(agg1, din_t, dsrc_t, b1_2, W2_p):
    return pl.pallas_call(
        _tc_mid_body,
        grid=(_GRID,),
        in_specs=[
            pl.BlockSpec((NC, _BM, D_HID), lambda i: (0, i, 0)),
            pl.BlockSpec((_BM, NC), lambda i: (i, 0)),
            pl.BlockSpec((_BM, NC), lambda i: (i, 0)),
            pl.BlockSpec((1, D_HID), lambda i: (0, 0)),
            pl.BlockSpec((D_HID, N_CLS_PAD), lambda i: (0, 0)),
        ],
        out_specs=pl.BlockSpec((_BM, N_CLS_PAD), lambda i: (i, 0)),
        out_shape=jax.ShapeDtypeStruct((N_PAD, N_CLS_PAD), jnp.float32),
    )(agg1, din_t, dsrc_t, b1_2, W2_p)


def _tc_post_body(a_ref, din_ref, b2_ref, o_ref):
    agg = a_ref[0] + a_ref[1]
    o_ref[...] = agg * _norm_col(din_ref[...]) + b2_ref[...]


def _tc_post(agg2, din_t, b2_2):
    return pl.pallas_call(
        _tc_post_body,
        grid=(_GRID,),
        in_specs=[
            pl.BlockSpec((NC, _BM, N_CLS_PAD), lambda i: (0, i, 0)),
            pl.BlockSpec((_BM, NC), lambda i: (i, 0)),
            pl.BlockSpec((1, N_CLS_PAD), lambda i: (0, 0)),
        ],
        out_specs=pl.BlockSpec((_BM, N_CLS_PAD), lambda i: (i, 0)),
        out_shape=jax.ShapeDtypeStruct((N_PAD, N_CLS_PAD), jnp.float32),
    )(agg2, din_t, b2_2)


def kernel(h, edge_index, W1, b1, W2, b2):
    pad = jnp.full((E_PAD - N_EDGES,), N_NODES, dtype=jnp.int32)
    src_p = jnp.concatenate([edge_index[0], pad]).reshape(N_BLOCKS_TOTAL, BLK)
    dst_p = jnp.concatenate([edge_index[1], pad]).reshape(N_BLOCKS_TOTAL, BLK)
    h_p = jnp.pad(h, ((0, N_PAD - N_NODES), (0, 0)))
    W2_p = jnp.pad(W2, ((0, 0), (0, N_CLS_PAD - N_CLS)))
    b1_2 = b1.reshape(1, D_HID)
    b2_2 = jnp.pad(b2, (0, N_CLS_PAD - N_CLS)).reshape(1, N_CLS_PAD)
    z1 = jnp.zeros((N_PAD,), jnp.float32)
    z128 = jnp.zeros((N_PAD, D_HID), jnp.float32)
    z48 = jnp.zeros((N_PAD, N_CLS_PAD), jnp.float32)

    dsrc, ddst = _sc_degrees(src_p, dst_p, z1)
    dsrc_t = dsrc.T
    ddst_t = ddst.T

    hw1 = _tc_pre(h_p, dsrc_t, W1)
    agg1 = _sc_aggregate_128(hw1, src_p, dst_p, z128)
    hw2 = _tc_mid(agg1, ddst_t, dsrc_t, b1_2, W2_p)
    agg2 = _sc_aggregate_48(hw2, src_p, dst_p, z48)
    out_p = _tc_post(agg2, ddst_t, b2_2)

    out = out_p[:N_NODES, :N_CLS]
    return (out, out)


# speedup vs baseline: 4.3566x; 1.1181x over previous
import functools

import jax
import jax.numpy as jnp
from jax import lax
from jax.experimental import pallas as pl
from jax.experimental.pallas import tpu as pltpu
from jax.experimental.pallas import tpu_sc as plsc

N_NODES = 10000
N_EDGES = 320000
D_IN = 128
D_HID = 128
N_CLS = 40
N_CLS_PAD = 48

NC = 2
NS = 16
NW = NC * NS
BLK = 128
NB = 80
E_PAD = NW * NB * BLK
N_BLOCKS_TOTAL = NW * NB
N_PAD = 10240
ROWS_PER_TILE = N_PAD // NS

_MESH = plsc.VectorSubcoreMesh(core_axis_name="c", subcore_axis_name="s")


@functools.partial(
    pl.kernel,
    out_type=(
        jax.ShapeDtypeStruct((NC, N_PAD), jnp.float32),
        jax.ShapeDtypeStruct((NC, N_PAD), jnp.float32),
    ),
    mesh=_MESH,
    scratch_types=[
        pltpu.VMEM((NB, BLK), jnp.int32),
        pltpu.VMEM((NB, BLK), jnp.int32),
        pltpu.VMEM((BLK,), jnp.float32),
        pltpu.VMEM_SHARED((N_PAD,), jnp.float32),
        pltpu.VMEM_SHARED((N_PAD,), jnp.float32),
    ],
)
def _sc_degrees(src_hbm, dst_hbm, zeros_hbm, dsrc_out, ddst_out,
                sidx, didx, ones, acc_s, acc_d):
    c = lax.axis_index("c")
    s = lax.axis_index("s")
    w = c * NS + s
    for k in range(BLK // 16):
        ones[pl.ds(k * 16, 16)] = jnp.ones((16,), jnp.float32)
    r0 = s * ROWS_PER_TILE
    pltpu.sync_copy(zeros_hbm.at[pl.ds(r0, ROWS_PER_TILE)],
                    acc_s.at[pl.ds(r0, ROWS_PER_TILE)])
    pltpu.sync_copy(zeros_hbm.at[pl.ds(r0, ROWS_PER_TILE)],
                    acc_d.at[pl.ds(r0, ROWS_PER_TILE)])
    pltpu.sync_copy(src_hbm.at[pl.ds(w * NB, NB)], sidx)
    pltpu.sync_copy(dst_hbm.at[pl.ds(w * NB, NB)], didx)
    plsc.subcore_barrier()

    def body(j, carry):
        pltpu.sync_copy(ones, acc_s.at[sidx.at[j]], add=True)
        pltpu.sync_copy(ones, acc_d.at[didx.at[j]], add=True)
        return carry

    lax.fori_loop(0, NB, body, 0)
    plsc.subcore_barrier()
    pltpu.sync_copy(acc_s.at[pl.ds(r0, ROWS_PER_TILE)],
                    dsrc_out.at[c, pl.ds(r0, ROWS_PER_TILE)])
    pltpu.sync_copy(acc_d.at[pl.ds(r0, ROWS_PER_TILE)],
                    ddst_out.at[c, pl.ds(r0, ROWS_PER_TILE)])


def _make_sc_aggregate(d_width):
    @functools.partial(
        pl.kernel,
        out_type=jax.ShapeDtypeStruct((NC, N_PAD, d_width), jnp.float32),
        mesh=_MESH,
        scratch_types=[
            pltpu.VMEM((NB // 2, BLK), jnp.int32),
            pltpu.VMEM((NB // 2, BLK), jnp.int32),
            pltpu.VMEM((BLK, d_width), jnp.float32),
            pltpu.VMEM((BLK, d_width), jnp.float32),
            pltpu.VMEM_SHARED((N_PAD, d_width), jnp.float32),
            pltpu.SemaphoreType.DMA,
            pltpu.SemaphoreType.DMA,
        ],
        compiler_params=pltpu.CompilerParams(
            use_tc_tiling_on_sc=(d_width % 128 == 0)),
    )
    def _sc_aggregate(vals_hbm, src_hbm, dst_hbm, zeros_hbm, out,
                      sidx, didx, buf0, buf1, acc, sem0, sem1):
        c = lax.axis_index("c")
        s = lax.axis_index("s")
        w = c * NS + s
        r0 = s * ROWS_PER_TILE
        pltpu.sync_copy(zeros_hbm.at[pl.ds(r0, ROWS_PER_TILE)],
                        acc.at[pl.ds(r0, ROWS_PER_TILE)])
        plsc.subcore_barrier()

        nh = NB // 2
        for half in range(2):
            base = w * NB + half * nh
            pltpu.sync_copy(src_hbm.at[pl.ds(base, nh)], sidx)
            pltpu.sync_copy(dst_hbm.at[pl.ds(base, nh)], didx)
            pltpu.async_copy(vals_hbm.at[sidx.at[0]], buf0, sem0)

            def body(p, carry):
                j = 2 * p
                pltpu.async_copy(vals_hbm.at[sidx.at[j + 1]], buf1, sem1)
                pltpu.make_async_copy(vals_hbm.at[sidx.at[j]], buf0,
                                      sem0).wait()
                pltpu.sync_copy(buf0, acc.at[didx.at[j]], add=True)

                @pl.when(p + 1 < nh // 2)
                def _():
                    pltpu.async_copy(vals_hbm.at[sidx.at[j + 2]], buf0, sem0)

                pltpu.make_async_copy(vals_hbm.at[sidx.at[j + 1]], buf1,
                                      sem1).wait()
                pltpu.sync_copy(buf1, acc.at[didx.at[j + 1]], add=True)
                return carry

            lax.fori_loop(0, nh // 2, body, 0)
        plsc.subcore_barrier()
        pltpu.sync_copy(acc.at[pl.ds(r0, ROWS_PER_TILE)],
                        out.at[c, pl.ds(r0, ROWS_PER_TILE)])

    return _sc_aggregate


_sc_aggregate_128 = _make_sc_aggregate(D_HID)
_sc_aggregate_48 = _make_sc_aggregate(N_CLS_PAD)


_BM = 1024
_GRID = N_PAD // _BM


def _norm_col(deg2):
    d = deg2[:, 0:1] + deg2[:, 1:2]
    return jnp.where(d > 0.0, lax.rsqrt(d), 0.0)


def _tc_pre_body(h_ref, dsrc_ref, w1_ref, o_ref):
    x = h_ref[...] * _norm_col(dsrc_ref[...])
    o_ref[...] = jnp.dot(x, w1_ref[...], preferred_element_type=jnp.float32)


def _tc_pre(h_p, dsrc_t, W1):
    return pl.pallas_call(
        _tc_pre_body,
        grid=(_GRID,),
        in_specs=[
            pl.BlockSpec((_BM, D_IN), lambda i: (i, 0)),
            pl.BlockSpec((_BM, NC), lambda i: (i, 0)),
            pl.BlockSpec((D_IN, D_HID), lambda i: (0, 0)),
        ],
        out_specs=pl.BlockSpec((_BM, D_HID), lambda i: (i, 0)),
        out_shape=jax.ShapeDtypeStruct((N_PAD, D_HID), jnp.float32),
    )(h_p, dsrc_t, W1)


def _tc_mid_body(a_ref, din_ref, dsrc_ref, b1_ref, w2_ref, o_ref):
    agg = a_ref[0] + a_ref[1]
    h1 = jnp.maximum(agg * _norm_col(din_ref[...]) + b1_ref[...], 0.0)
    x = h1 * _norm_col(dsrc_ref[...])
    o_ref[...] = jnp.dot(x, w2_ref[...], preferred_element_type=jnp.float32)


def _tc_mid(agg1, din_t, dsrc_t, b1_2, W2_p):
    return pl.pallas_call(
        _tc_mid_body,
        grid=(_GRID,),
        in_specs=[
            pl.BlockSpec((NC, _BM, D_HID), lambda i: (0, i, 0)),
            pl.BlockSpec((_BM, NC), lambda i: (i, 0)),
            pl.BlockSpec((_BM, NC), lambda i: (i, 0)),
            pl.BlockSpec((1, D_HID), lambda i: (0, 0)),
            pl.BlockSpec((D_HID, N_CLS_PAD), lambda i: (0, 0)),
        ],
        out_specs=pl.BlockSpec((_BM, N_CLS_PAD), lambda i: (i, 0)),
        out_shape=jax.ShapeDtypeStruct((N_PAD, N_CLS_PAD), jnp.float32),
    )(agg1, din_t, dsrc_t, b1_2, W2_p)


def _tc_post_body(a_ref, din_ref, b2_ref, o_ref):
    agg = a_ref[0] + a_ref[1]
    o_ref[...] = agg * _norm_col(din_ref[...]) + b2_ref[...]


def _tc_post(agg2, din_t, b2_2):
    return pl.pallas_call(
        _tc_post_body,
        grid=(_GRID,),
        in_specs=[
            pl.BlockSpec((NC, _BM, N_CLS_PAD), lambda i: (0, i, 0)),
            pl.BlockSpec((_BM, NC), lambda i: (i, 0)),
            pl.BlockSpec((1, N_CLS_PAD), lambda i: (0, 0)),
        ],
        out_specs=pl.BlockSpec((_BM, N_CLS_PAD), lambda i: (i, 0)),
        out_shape=jax.ShapeDtypeStruct((N_PAD, N_CLS_PAD), jnp.float32),
    )(agg2, din_t, b2_2)


def kernel(h, edge_index, W1, b1, W2, b2):
    pad = jnp.full((E_PAD - N_EDGES,), N_NODES, dtype=jnp.int32)
    src_p = jnp.concatenate([edge_index[0], pad]).reshape(N_BLOCKS_TOTAL, BLK)
    dst_p = jnp.concatenate([edge_index[1], pad]).reshape(N_BLOCKS_TOTAL, BLK)
    h_p = jnp.pad(h, ((0, N_PAD - N_NODES), (0, 0)))
    W2_p = jnp.pad(W2, ((0, 0), (0, N_CLS_PAD - N_CLS)))
    b1_2 = b1.reshape(1, D_HID)
    b2_2 = jnp.pad(b2, (0, N_CLS_PAD - N_CLS)).reshape(1, N_CLS_PAD)
    z1 = jnp.zeros((N_PAD,), jnp.float32)
    z128 = jnp.zeros((N_PAD, D_HID), jnp.float32)
    z48 = jnp.zeros((N_PAD, N_CLS_PAD), jnp.float32)

    dsrc, ddst = _sc_degrees(src_p, dst_p, z1)
    dsrc_t = dsrc.T
    ddst_t = ddst.T

    hw1 = _tc_pre(h_p, dsrc_t, W1)
    agg1 = _sc_aggregate_128(hw1, src_p, dst_p, z128)
    hw2 = _tc_mid(agg1, ddst_t, dsrc_t, b1_2, W2_p)
    agg2 = _sc_aggregate_48(hw2, src_p, dst_p, z48)
    out_p = _tc_post(agg2, ddst_t, b2_2)

    out = out_p[:N_NODES, :N_CLS]
    return (out, out)
